# baseline (speedup 1.0000x reference)
import functools

import jax
import jax.numpy as jnp
from jax import lax
from jax.experimental import pallas as pl
from jax.experimental.pallas import tpu as pltpu
from jax.experimental.pallas import tpu_sc as plsc

D = 64
SCALE = 8.0
IW = 128
NC = 2
NS = 16
NW = NC * NS
K = 8
CHUNK = K * IW


@functools.partial(jax.jit, static_argnames=("T", "B"))
def _emb_lookup(t128, xt3, *, T, B):
    blocks_per_t = B // CHUNK
    n_chunks_total = T * blocks_per_t
    chunks_per_w = n_chunks_total // NW

    mesh = plsc.VectorSubcoreMesh(core_axis_name="c", subcore_axis_name="s")

    @functools.partial(
        pl.kernel,
        mesh=mesh,
        compiler_params=pltpu.CompilerParams(
            use_tc_tiling_on_sc=False, needs_layout_passes=False
        ),
        out_type=jax.ShapeDtypeStruct(
            (T, D // 8, B // 128, 8, 128), jnp.float32
        ),
        scratch_types=[
            pltpu.VMEM((K, IW), jnp.int32),
            pltpu.VMEM((K, IW), jnp.int32),
            pltpu.VMEM((IW, 2 * D), jnp.float32),
            pltpu.VMEM((IW, 2 * D), jnp.float32),
            pltpu.VMEM((D // 8, K, 8, IW), jnp.float32),
            pltpu.SemaphoreType.DMA,
            pltpu.SemaphoreType.DMA,
        ],
    )
    def body(t_hbm, idx_hbm, out_hbm, idx_v, jdx_v, pair0, pair1, out_v,
             semA, semB):
        wid = lax.axis_index("s") * NC + lax.axis_index("c")
        cid0 = wid * chunks_per_w
        sems = (semA, semB)
        pair_bufs = (pair0, pair1)
        lane = lax.iota(jnp.int32, 16)

        def select_group(g, buf):
            pv = pair_bufs[buf]

            def sel_block(blk, carry):
                rows = blk * 16 + lane
                cols0 = (idx_v[g, pl.ds(blk * 16, 16)] & 1) * D
                for d in range(D):
                    vals = plsc.load_gather(pv, [rows, cols0 + d]) * SCALE
                    out_v[d // 8, g, d % 8, pl.ds(blk * 16, 16)] = vals
                return carry

            lax.fori_loop(0, IW // 16, sel_block, 0)

        def chunk_body(c, carry):
            cid = cid0 + c
            tpos = cid // blocks_per_t
            hb = cid % blocks_per_t
            pltpu.sync_copy(idx_hbm.at[tpos, pl.ds(hb * K, K)], idx_v)
            for r in range(K):
                for s in range(IW // 16):
                    jdx_v[r, pl.ds(s * 16, 16)] = (
                        idx_v[r, pl.ds(s * 16, 16)] >> 1
                    )
            cps = [None] * K
            cps[0] = pltpu.async_copy(
                t_hbm.at[jdx_v.at[0]], pair_bufs[0], sems[0]
            )
            for g in range(1, K):
                cps[g] = pltpu.async_copy(
                    t_hbm.at[jdx_v.at[g]], pair_bufs[g % 2], sems[g % 2]
                )
                cps[g - 1].wait()
                select_group(g - 1, (g - 1) % 2)
            cps[K - 1].wait()
            select_group(K - 1, (K - 1) % 2)

            pltpu.sync_copy(
                out_v, out_hbm.at[tpos, :, pl.ds(hb * K, K)]
            )
            return carry

        lax.fori_loop(0, chunks_per_w, chunk_body, 0)

    return body(t128, xt3)


def kernel(x, table):
    b, t = x.shape
    xt3 = x.astype(jnp.int32).T.reshape(t, b // IW, IW)
    t128 = jnp.concatenate([table[0::2], table[1::2]], axis=1)
    out5 = _emb_lookup(t128, xt3, T=t, B=b)
    return out5.transpose(2, 4, 0, 1, 3).reshape(b, t, D)


# speedup vs baseline: 6.5841x; 6.5841x over previous
import functools

import jax
import jax.numpy as jnp
from jax import lax
from jax.experimental import pallas as pl
from jax.experimental.pallas import tpu as pltpu
from jax.experimental.pallas import tpu_sc as plsc

D = 64
SCALE = 8.0
NC = 2
NS = 16
NW = NC * NS
C = 256
NBUF = 4


@functools.partial(jax.jit, static_argnames=("n",))
def _emb_lookup(t_rm, idx3, *, n):
    per_w = n // NW
    n_chunks = per_w // C

    mesh = plsc.VectorSubcoreMesh(core_axis_name="c", subcore_axis_name="s")

    @functools.partial(
        pl.kernel,
        mesh=mesh,
        compiler_params=pltpu.CompilerParams(
            use_tc_tiling_on_sc=False, needs_layout_passes=False
        ),
        out_type=jax.ShapeDtypeStruct((n // C, C, D), jnp.float32),
        scratch_types=[
            pltpu.VMEM((n_chunks, C), jnp.int32),
            pltpu.VMEM((NBUF, C, D), jnp.float32),
        ]
        + [pltpu.SemaphoreType.DMA] * (2 * NBUF),
    )
    def body(t_hbm, idx_hbm, out_hbm, idx_v, bufs, *sems):
        gsem = sems[:NBUF]
        osem = sems[NBUF:]
        wid = lax.axis_index("s") * NC + lax.axis_index("c")
        chunk0 = wid * n_chunks

        pltpu.sync_copy(idx_hbm.at[pl.ds(chunk0, n_chunks)], idx_v)

        def gather(q):
            b = q % NBUF
            return pltpu.async_copy(
                t_hbm.at[idx_v.at[q]], bufs.at[b], gsem[b]
            )

        def drain(q):
            b = q % NBUF
            return pltpu.async_copy(
                bufs.at[b], out_hbm.at[chunk0 + q], osem[b]
            )

        gh = [None] * n_chunks
        oh = [None] * n_chunks
        for b in range(NBUF):
            gh[b] = gather(b)
        for q in range(n_chunks):
            gh[q].wait()
            oh[q] = drain(q)
            if q + NBUF < n_chunks:
                oh[q].wait()
                gh[q + NBUF] = gather(q + NBUF)
        for q in range(n_chunks - NBUF, n_chunks):
            oh[q].wait()

    return body(t_rm, idx3)


def kernel(x, table):
    b, t = x.shape
    n = b * t
    idx3 = x.astype(jnp.int32).reshape(n // C, C)
    t_rm = table * SCALE
    out = _emb_lookup(t_rm, idx3, n=n)
    return out.reshape(b, t, D)


# speedup vs baseline: 8.2239x; 1.2491x over previous
import functools

import jax
import jax.numpy as jnp
from jax import lax
from jax.experimental import pallas as pl
from jax.experimental.pallas import tpu as pltpu
from jax.experimental.pallas import tpu_sc as plsc

D = 64
SCALE = 8.0
NC = 2
NS = 16
NW = NC * NS
C = 256
NBUF = 4


@functools.partial(jax.jit, static_argnames=("n",))
def _emb_lookup(t_rm, idx3, *, n):
    per_w = n // NW
    n_chunks = per_w // C

    mesh = plsc.VectorSubcoreMesh(core_axis_name="c", subcore_axis_name="s")

    @functools.partial(
        pl.kernel,
        mesh=mesh,
        compiler_params=pltpu.CompilerParams(
            use_tc_tiling_on_sc=False, needs_layout_passes=False
        ),
        out_type=jax.ShapeDtypeStruct((n // C, C, D), jnp.float32),
        scratch_types=[
            pltpu.VMEM((n_chunks, C), jnp.int32),
            pltpu.VMEM((NBUF, C, D), jnp.float32),
        ]
        + [pltpu.SemaphoreType.DMA] * (2 * NBUF),
    )
    def body(t_hbm, idx_hbm, out_hbm, idx_v, bufs, *sems):
        gsem = sems[:NBUF]
        osem = sems[NBUF:]
        wid = lax.axis_index("s") * NC + lax.axis_index("c")
        chunk0 = wid * n_chunks

        pltpu.sync_copy(idx_hbm.at[pl.ds(chunk0, n_chunks)], idx_v)

        def gather(q):
            b = q % NBUF
            return pltpu.async_copy(
                t_hbm.at[idx_v.at[q]], bufs.at[b], gsem[b]
            )

        def scale(q):
            b = q % NBUF

            def row_body(i, carry):
                row = bufs.at[b].at[i]
                for j in range(D // 16):
                    s = pl.ds(j * 16, 16)
                    row[s] = row[s] * SCALE
                return carry

            lax.fori_loop(0, C, row_body, 0, unroll=4)

        def drain(q):
            b = q % NBUF
            return pltpu.async_copy(
                bufs.at[b], out_hbm.at[chunk0 + q], osem[b]
            )

        gh = [None] * n_chunks
        oh = [None] * n_chunks
        for b in range(NBUF):
            gh[b] = gather(b)
        for q in range(n_chunks):
            gh[q].wait()
            scale(q)
            oh[q] = drain(q)
            if q + NBUF < n_chunks:
                oh[q].wait()
                gh[q + NBUF] = gather(q + NBUF)
        for q in range(n_chunks - NBUF, n_chunks):
            oh[q].wait()

    return body(t_rm, idx3)


def kernel(x, table):
    b, t = x.shape
    n = b * t
    idx3 = x.astype(jnp.int32).reshape(n // C, C)
    out = _emb_lookup(table, idx3, n=n)
    return out.reshape(b, t, D)
